# baseline (speedup 1.0000x reference)
import functools
import math

import jax
import jax.numpy as jnp
from jax import lax
from jax.experimental import pallas as pl
from jax.experimental.pallas import tpu as pltpu
from jax.experimental.pallas import tpu_sc as plsc

_N = 2048
_NCLASSES = 32000
_PADDING_IDX = 0
_SMOOTHING = 0.1
_CONFIDENCE = 1.0 - _SMOOTHING
_EPS = _SMOOTHING / (_NCLASSES - 2)
_C_ROW = (_NCLASSES - 1) * _EPS * math.log(_EPS) + _CONFIDENCE * math.log(_CONFIDENCE)

_R_TC = 1024
_BR = 256
_BC = 6400


def _sc_gather_partials(ty_true, flat_prob):
    info = plsc.get_sparse_core_info()
    nc, ns, L = info.num_cores, info.num_subcores, info.num_lanes
    nw = nc * ns
    bpw = _N // nw

    mesh = plsc.VectorSubcoreMesh(core_axis_name="c", subcore_axis_name="s")

    @functools.partial(
        pl.kernel,
        mesh=mesh,
        out_type=jax.ShapeDtypeStruct((nw, L), jnp.float32),
        scratch_types=[
            pltpu.VMEM((bpw,), jnp.int32),
            pltpu.VMEM((bpw,), jnp.int32),
            pltpu.VMEM((bpw,), jnp.float32),
            pltpu.VMEM((L,), jnp.float32),
            pltpu.SemaphoreType.DMA,
        ],
    )
    def sc_kernel(ttrue_hbm, flat_hbm, out_hbm, t_v, idx_v, val_v, acc_v, sem):
        wid = lax.axis_index("c") * ns + lax.axis_index("s")
        base = wid * bpw
        pltpu.sync_copy(ttrue_hbm.at[pl.ds(base, bpw)], t_v)
        for c in range(bpw // L):
            t16 = t_v[pl.ds(c * L, L)]
            rows = base + c * L + lax.iota(jnp.int32, L)
            idx_v[pl.ds(c * L, L)] = rows * _NCLASSES + t16
        pltpu.async_copy(flat_hbm.at[idx_v], val_v, sem).wait()
        acc = jnp.zeros((L,), jnp.float32)
        for c in range(bpw // L):
            t16 = t_v[pl.ds(c * L, L)]
            v16 = val_v[pl.ds(c * L, L)]
            acc = acc + jnp.where(t16 != _PADDING_IDX, v16, 0.0)
        acc_v[...] = acc
        pltpu.sync_copy(acc_v, out_hbm.at[wid])

    return sc_kernel(ty_true, flat_prob)


def _sc_dense_partials(ty_true, ty_prob):
    info = plsc.get_sparse_core_info()
    nc, ns, L = info.num_cores, info.num_subcores, info.num_lanes
    nw = nc * ns
    r_sc = _N - _R_TC
    bpw = r_sc // nw
    nvec = _NCLASSES // L
    unroll = 16
    niter = nvec // unroll

    mesh = plsc.VectorSubcoreMesh(core_axis_name="c", subcore_axis_name="s")

    @functools.partial(
        pl.kernel,
        mesh=mesh,
        out_type=jax.ShapeDtypeStruct((nw, 2 * L), jnp.float32),
        scratch_types=[
            pltpu.VMEM((bpw,), jnp.int32),
            pltpu.VMEM((_NCLASSES,), jnp.float32),
            pltpu.VMEM((_NCLASSES,), jnp.float32),
            pltpu.VMEM((2 * L,), jnp.float32),
            pltpu.SemaphoreType.DMA,
            pltpu.SemaphoreType.DMA,
        ],
    )
    def sc_kernel(ttrue_hbm, prob_hbm, out_hbm, t_v, buf0, buf1, acc_v,
                  sem0, sem1):
        wid = lax.axis_index("c") * ns + lax.axis_index("s")
        base = _R_TC + wid * bpw
        pltpu.sync_copy(ttrue_hbm.at[pl.ds(base, bpw)], t_v)

        bufs = (buf0, buf1)
        sems = (sem0, sem1)
        pltpu.async_copy(prob_hbm.at[base], buf0, sem0)

        total = jnp.zeros((L,), jnp.float32)
        count = jnp.zeros((L,), jnp.float32)
        lanes = lax.iota(jnp.int32, L)
        for r in range(bpw):
            buf, sem = bufs[r % 2], sems[r % 2]
            pltpu.make_async_copy(prob_hbm.at[base + r], buf, sem).wait()
            if r + 1 < bpw:
                pltpu.async_copy(prob_hbm.at[base + r + 1],
                                 bufs[(r + 1) % 2], sems[(r + 1) % 2])

            def body(k, accs):
                o = k * (unroll * L)
                new = list(accs)
                for u in range(unroll):
                    v = buf[pl.ds(o + u * L, L)]
                    new[u % len(new)] = new[u % len(new)] + v
                return tuple(new)

            accs0 = tuple(jnp.zeros((L,), jnp.float32) for _ in range(8))
            accs = lax.fori_loop(0, niter, body, accs0)
            vec = accs[0]
            for a in accs[1:]:
                vec = vec + a

            for sh in (8, 4, 2, 1):
                vec = vec + vec.at[lanes ^ sh].get(mode="promise_in_bounds")

            t16 = t_v[pl.ds((r // L) * L, L)]
            here = lanes == (r % L)
            rowmask = here & (t16 != _PADDING_IDX)
            total = total + jnp.where(rowmask, vec, 0.0)
            count = count + jnp.where(rowmask, 1.0, 0.0)

        acc_v[pl.ds(0, L)] = total
        acc_v[pl.ds(L, L)] = count
        pltpu.sync_copy(acc_v, out_hbm.at[wid])

    return sc_kernel(ty_true, ty_prob)


def _tc_body(nr, ncol, prob_ref, ttrue_ref, out_ref, acc_ref):
    i = pl.program_id(0)
    j = pl.program_id(1)

    @pl.when((i == 0) & (j == 0))
    def _init():
        acc_ref[0] = 0.0
        acc_ref[1] = 0.0

    t = ttrue_ref[...]
    rowsum = prob_ref[...].sum(axis=1, keepdims=True)
    nonpad = t != _PADDING_IDX
    acc_ref[0] += jnp.sum(jnp.where(nonpad, rowsum, 0.0))

    @pl.when(j == 0)
    def _count():
        acc_ref[1] += jnp.sum(jnp.where(nonpad, 1.0, 0.0))

    @pl.when((i == nr - 1) & (j == ncol - 1))
    def _finish():
        out_ref[0, 0] = acc_ref[0]
        out_ref[0, 1] = acc_ref[1]


def kernel(ty_prob, ty_true):
    gather_part = _sc_gather_partials(ty_true, ty_prob.reshape(-1))
    dense_part = _sc_dense_partials(ty_true, ty_prob)

    nr = _R_TC // _BR
    ncol = _NCLASSES // _BC
    tc_out = pl.pallas_call(
        functools.partial(_tc_body, nr, ncol),
        grid=(nr, ncol),
        in_specs=[
            pl.BlockSpec((_BR, _BC), lambda i, j: (i, j)),
            pl.BlockSpec((_BR, 1), lambda i, j: (i, 0)),
        ],
        out_specs=pl.BlockSpec(memory_space=pltpu.SMEM),
        out_shape=jax.ShapeDtypeStruct((1, 2), jnp.float32),
        scratch_shapes=[pltpu.SMEM((2,), jnp.float32)],
        compiler_params=pltpu.CompilerParams(
            dimension_semantics=("arbitrary", "arbitrary")),
    )(ty_prob, ty_true.reshape(_N, 1))

    s2 = jnp.sum(gather_part)
    s1 = tc_out[0, 0] + jnp.sum(dense_part[:, :16])
    cnt = tc_out[0, 1] + jnp.sum(dense_part[:, 16:])
    return _C_ROW * cnt - _EPS * s1 - (_CONFIDENCE - _EPS) * s2


# speedup vs baseline: 1.0008x; 1.0008x over previous
import functools
import math

import jax
import jax.numpy as jnp
from jax import lax
from jax.experimental import pallas as pl
from jax.experimental.pallas import tpu as pltpu
from jax.experimental.pallas import tpu_sc as plsc

_N = 2048
_NCLASSES = 32000
_PADDING_IDX = 0
_SMOOTHING = 0.1
_CONFIDENCE = 1.0 - _SMOOTHING
_EPS = _SMOOTHING / (_NCLASSES - 2)
_C_ROW = (_NCLASSES - 1) * _EPS * math.log(_EPS) + _CONFIDENCE * math.log(_CONFIDENCE)

_R_TC = 1024
_BR = 256
_BC = 6400


def _sc_gather_partials(ty_true, flat_prob):
    info = plsc.get_sparse_core_info()
    nc, ns, L = info.num_cores, info.num_subcores, info.num_lanes
    nw = nc * ns
    bpw = _N // nw

    mesh = plsc.VectorSubcoreMesh(core_axis_name="c", subcore_axis_name="s")

    @functools.partial(
        pl.kernel,
        mesh=mesh,
        out_type=jax.ShapeDtypeStruct((nw, L), jnp.float32),
        scratch_types=[
            pltpu.VMEM((bpw,), jnp.int32),
            pltpu.VMEM((bpw,), jnp.int32),
            pltpu.VMEM((bpw,), jnp.float32),
            pltpu.VMEM((L,), jnp.float32),
            pltpu.SemaphoreType.DMA,
        ],
    )
    def sc_kernel(ttrue_hbm, flat_hbm, out_hbm, t_v, idx_v, val_v, acc_v, sem):
        wid = lax.axis_index("c") * ns + lax.axis_index("s")
        base = wid * bpw
        pltpu.sync_copy(ttrue_hbm.at[pl.ds(base, bpw)], t_v)
        for c in range(bpw // L):
            t16 = t_v[pl.ds(c * L, L)]
            rows = base + c * L + lax.iota(jnp.int32, L)
            idx_v[pl.ds(c * L, L)] = rows * _NCLASSES + t16
        pltpu.async_copy(flat_hbm.at[idx_v], val_v, sem).wait()
        acc = jnp.zeros((L,), jnp.float32)
        for c in range(bpw // L):
            t16 = t_v[pl.ds(c * L, L)]
            v16 = val_v[pl.ds(c * L, L)]
            acc = acc + jnp.where(t16 != _PADDING_IDX, v16, 0.0)
        acc_v[...] = acc
        pltpu.sync_copy(acc_v, out_hbm.at[wid])

    return sc_kernel(ty_true, flat_prob)


def _sc_dense_partials(ty_true, ty_prob):
    info = plsc.get_sparse_core_info()
    nc, ns, L = info.num_cores, info.num_subcores, info.num_lanes
    nw = nc * ns
    r_sc = _N - _R_TC
    bpw = r_sc // nw
    nvec = _NCLASSES // L
    unroll = 16
    niter = nvec // unroll

    mesh = plsc.VectorSubcoreMesh(core_axis_name="c", subcore_axis_name="s")

    @functools.partial(
        pl.kernel,
        mesh=mesh,
        out_type=jax.ShapeDtypeStruct((nw, 2 * L), jnp.float32),
        scratch_types=[
            pltpu.VMEM((bpw,), jnp.int32),
            pltpu.VMEM((_NCLASSES,), jnp.float32),
            pltpu.VMEM((_NCLASSES,), jnp.float32),
            pltpu.VMEM((2 * L,), jnp.float32),
            pltpu.SemaphoreType.DMA,
            pltpu.SemaphoreType.DMA,
        ],
    )
    def sc_kernel(ttrue_hbm, prob_hbm, out_hbm, t_v, buf0, buf1, acc_v,
                  sem0, sem1):
        wid = lax.axis_index("c") * ns + lax.axis_index("s")
        base = _R_TC + wid * bpw
        pltpu.sync_copy(ttrue_hbm.at[pl.ds(base, bpw)], t_v)

        bufs = (buf0, buf1)
        sems = (sem0, sem1)
        pltpu.async_copy(prob_hbm.at[base], buf0, sem0)

        total = jnp.zeros((L,), jnp.float32)
        count = jnp.zeros((L,), jnp.float32)
        lanes = lax.iota(jnp.int32, L)
        for r in range(bpw):
            buf, sem = bufs[r % 2], sems[r % 2]
            pltpu.make_async_copy(prob_hbm.at[base + r], buf, sem).wait()
            if r + 1 < bpw:
                pltpu.async_copy(prob_hbm.at[base + r + 1],
                                 bufs[(r + 1) % 2], sems[(r + 1) % 2])

            accs0 = tuple(jnp.zeros((L,), jnp.float32) for _ in range(8))

            @plsc.parallel_loop(0, niter, carry=accs0, unroll=5)
            def accs(k, accs):
                o = k * (unroll * L)
                new = list(accs)
                for u in range(unroll):
                    v = buf[pl.ds(o + u * L, L)]
                    new[u % len(new)] = new[u % len(new)] + v
                return tuple(new)
            vec = accs[0]
            for a in accs[1:]:
                vec = vec + a

            for sh in (8, 4, 2, 1):
                vec = vec + vec.at[lanes ^ sh].get(mode="promise_in_bounds")

            t16 = t_v[pl.ds((r // L) * L, L)]
            here = lanes == (r % L)
            rowmask = here & (t16 != _PADDING_IDX)
            total = total + jnp.where(rowmask, vec, 0.0)
            count = count + jnp.where(rowmask, 1.0, 0.0)

        acc_v[pl.ds(0, L)] = total
        acc_v[pl.ds(L, L)] = count
        pltpu.sync_copy(acc_v, out_hbm.at[wid])

    return sc_kernel(ty_true, ty_prob)


def _tc_body(nr, ncol, prob_ref, ttrue_ref, out_ref, acc_ref):
    i = pl.program_id(0)
    j = pl.program_id(1)

    @pl.when((i == 0) & (j == 0))
    def _init():
        acc_ref[0] = 0.0
        acc_ref[1] = 0.0

    t = ttrue_ref[...]
    rowsum = prob_ref[...].sum(axis=1, keepdims=True)
    nonpad = t != _PADDING_IDX
    acc_ref[0] += jnp.sum(jnp.where(nonpad, rowsum, 0.0))

    @pl.when(j == 0)
    def _count():
        acc_ref[1] += jnp.sum(jnp.where(nonpad, 1.0, 0.0))

    @pl.when((i == nr - 1) & (j == ncol - 1))
    def _finish():
        out_ref[0, 0] = acc_ref[0]
        out_ref[0, 1] = acc_ref[1]


def kernel(ty_prob, ty_true):
    gather_part = _sc_gather_partials(ty_true, ty_prob.reshape(-1))
    dense_part = _sc_dense_partials(ty_true, ty_prob)

    nr = _R_TC // _BR
    ncol = _NCLASSES // _BC
    tc_out = pl.pallas_call(
        functools.partial(_tc_body, nr, ncol),
        grid=(nr, ncol),
        in_specs=[
            pl.BlockSpec((_BR, _BC), lambda i, j: (i, j)),
            pl.BlockSpec((_BR, 1), lambda i, j: (i, 0)),
        ],
        out_specs=pl.BlockSpec(memory_space=pltpu.SMEM),
        out_shape=jax.ShapeDtypeStruct((1, 2), jnp.float32),
        scratch_shapes=[pltpu.SMEM((2,), jnp.float32)],
        compiler_params=pltpu.CompilerParams(
            dimension_semantics=("arbitrary", "arbitrary")),
    )(ty_prob, ty_true.reshape(_N, 1))

    s2 = jnp.sum(gather_part)
    s1 = tc_out[0, 0] + jnp.sum(dense_part[:, :16])
    cnt = tc_out[0, 1] + jnp.sum(dense_part[:, 16:])
    return _C_ROW * cnt - _EPS * s1 - (_CONFIDENCE - _EPS) * s2
